# Initial kernel scaffold; baseline (speedup 1.0000x reference)
import jax
import jax.numpy as jnp
from jax import lax
from jax.experimental import pallas as pl
from jax.experimental.pallas import tpu as pltpu
from jax.experimental.pallas import tpu_sc as plsc

NC, NS = 2, 16
NW = NC * NS
NROW = 1024
DUMMY = 1008
D = 128
CH = 128
RPT = NROW // NS


def _make_sc_agg(E, filter_dst):
    EPW = E // NW
    NCH = (EPW + CH - 1) // CH
    KP = NCH * CH

    mesh = plsc.VectorSubcoreMesh(
        core_axis_name="c", subcore_axis_name="s",
        num_cores=NC, num_subcores=NS)
    out_type = (
        jax.ShapeDtypeStruct((NC, NROW, D), jnp.float32),
        jax.ShapeDtypeStruct((NC, NS, NROW), jnp.float32),
    )
    scratch = []
    if filter_dst:
        scratch += [pltpu.VMEM((EPW,), jnp.int32),
                    pltpu.VMEM((EPW,), jnp.int32)]
    scratch += [
        pltpu.VMEM((KP,), jnp.int32),
        pltpu.VMEM((KP,), jnp.int32),
        pltpu.VMEM((CH,), jnp.int32),
        pltpu.VMEM((CH,), jnp.int32),
        pltpu.VMEM((CH, D), jnp.float32),
        pltpu.VMEM((NROW,), jnp.float32),
        pltpu.VMEM_SHARED((NROW, D), jnp.float32),
        pltpu.SemaphoreType.DMA,
    ]

    def body(src_hbm, dst_hbm, table_hbm, sum_out, cnt_out, *sc):
        if filter_dst:
            src_loc, dst_loc = sc[0], sc[1]
            sc = sc[2:]
        src_keep, dst_keep, src_stage, dst_stage, rows, cnt_loc, acc, sem = sc
        cid = lax.axis_index("c")
        sid = lax.axis_index("s")
        wid = cid * NS + sid
        zf = jnp.zeros((16,), jnp.float32)
        zi = jnp.zeros((16,), jnp.int32)
        ones = jnp.ones((16,), jnp.float32)
        dums = jnp.full((16,), DUMMY, jnp.int32)
        iota = lax.iota(jnp.int32, 16)

        @pl.loop(0, RPT)
        def _zero(i):
            cnt_loc[pl.ds(i * 16, 16)] = zf
            for j in range(D // 16):
                rows[i, pl.ds(j * 16, 16)] = zf

        pltpu.sync_copy(rows.at[pl.ds(0, RPT)], acc.at[pl.ds(sid * RPT, RPT)])

        base = wid * EPW
        if filter_dst:
            pltpu.sync_copy(src_hbm.at[pl.ds(base, EPW)], src_loc)
            pltpu.sync_copy(dst_hbm.at[pl.ds(base, EPW)], dst_loc)
        else:
            pltpu.sync_copy(src_hbm.at[pl.ds(base, EPW)], src_keep.at[pl.ds(0, EPW)])
            pltpu.sync_copy(dst_hbm.at[pl.ds(base, EPW)], dst_keep.at[pl.ds(0, EPW)])

        plsc.subcore_barrier()

        if filter_dst:
            def cbody(i, ptr):
                s16 = src_loc[pl.ds(i * 16, 16)]
                d16 = dst_loc[pl.ds(i * 16, 16)]
                keep = d16 < 1000
                cs = plsc.cumsum(keep.astype(jnp.int32))
                pos = jnp.maximum(ptr + cs - 1, 0)
                plsc.store_scatter(src_keep, [pos], s16, mask=keep)
                plsc.store_scatter(dst_keep, [pos], d16, mask=keep)
                dsafe = jnp.where(keep, d16, DUMMY)
                plsc.addupdate_scatter(cnt_loc, [dsafe], ones, mask=keep)
                return ptr + jnp.max(cs)

            nkeep = pl.loop(0, EPW // 16, init_carry=jnp.int32(0))(cbody)
        else:
            nkeep = EPW

        base0 = (nkeep // CH) * CH
        for j in range(CH // 16):
            idx = base0 + j * 16 + iota
            m = idx >= nkeep
            plsc.store_scatter(src_keep, [idx], zi, mask=m)
            plsc.store_scatter(dst_keep, [idx], dums, mask=m)

        if not filter_dst:
            @pl.loop(0, (EPW + 15) // 16)
            def _hist(i):
                d16 = dst_keep[pl.ds(i * 16, 16)]
                plsc.addupdate_scatter(cnt_loc, [d16], ones)

        nch = (nkeep + CH - 1) // CH

        @pl.loop(0, nch)
        def _chunk(g):
            for j in range(CH // 16):
                src_stage[pl.ds(j * 16, 16)] = src_keep[pl.ds(g * CH + j * 16, 16)]
                dst_stage[pl.ds(j * 16, 16)] = dst_keep[pl.ds(g * CH + j * 16, 16)]
            pltpu.async_copy(table_hbm.at[src_stage], rows, sem).wait()
            pltpu.sync_copy(rows, acc.at[dst_stage], add=True)

        plsc.subcore_barrier()
        pltpu.sync_copy(acc.at[pl.ds(sid * RPT, RPT)],
                        sum_out.at[cid, pl.ds(sid * RPT, RPT)])
        pltpu.sync_copy(cnt_loc, cnt_out.at[cid, sid])

    return pl.kernel(body, out_type=out_type, mesh=mesh, scratch_types=scratch)


_E0, _E1 = 320000, 160000
_agg_l0 = _make_sc_agg(_E0, filter_dst=True)
_agg_l1 = _make_sc_agg(_E1, filter_dst=False)


def _dense0_body(sums, cnts, xd, wl, b, wr, out):
    s = sums[0] + sums[1]
    c = jnp.sum(cnts[...], axis=0)
    agg = s * (1.0 / jnp.maximum(c, 1.0))[:, None]
    h = jnp.dot(agg, wl[...], preferred_element_type=jnp.float32)
    h = h + b[...] + jnp.dot(xd[...], wr[...], preferred_element_type=jnp.float32)
    out[...] = jnp.maximum(h, 0.0)


def _dense1_body(sums, cnts, h0, wl, b, wr, wlin, blin, l0, l1, out):
    s = sums[0] + sums[1]
    c = jnp.sum(cnts[...], axis=0)
    agg = s * (1.0 / jnp.maximum(c, 1.0))[:, None]
    h = jnp.dot(agg, wl[...], preferred_element_type=jnp.float32)
    h = h + b[...] + jnp.dot(h0[...], wr[...], preferred_element_type=jnp.float32)
    h = jnp.maximum(h, 0.0)
    w2 = wlin[...]
    p0 = jnp.sum(h * w2[0:1, :], axis=1)
    p1 = jnp.sum(h * w2[1:2, :], axis=1)
    cols = lax.broadcasted_iota(jnp.int32, (512, NROW), 1)
    m0 = cols == l0[...]
    m1 = cols == l1[...]
    res = (jnp.sum(jnp.where(m0, p0[None, :], 0.0), axis=1)
           + jnp.sum(jnp.where(m1, p1[None, :], 0.0), axis=1))
    out[...] = res + blin[...][0]


def kernel(x, edge_index_0, edge_index_1, link, n_id,
           W0_l, b0_l, W0_r, W1_l, b1_l, W1_r, W_lin, b_lin):
    f32 = jnp.float32
    sums0, cnts0 = _agg_l0(edge_index_0[0], edge_index_0[1], x)
    h0 = pl.pallas_call(
        _dense0_body,
        out_shape=jax.ShapeDtypeStruct((NROW, D), f32),
    )(sums0, cnts0.reshape(NW, NROW), x[:NROW], W0_l,
      b0_l.reshape(1, D), W0_r)
    sums1, cnts1 = _agg_l1(edge_index_1[0], edge_index_1[1], h0)
    out = pl.pallas_call(
        _dense1_body,
        out_shape=jax.ShapeDtypeStruct((512,), f32),
    )(sums1, cnts1.reshape(NW, NROW), h0, W1_l,
      b1_l.reshape(1, D), W1_r, W_lin[:, 0].reshape(2, D),
      b_lin.reshape(1,), link[:, 0].reshape(512, 1), link[:, 1].reshape(512, 1))
    return out


# speedup vs baseline: 7.7235x; 7.7235x over previous
import jax
import jax.numpy as jnp
from jax import lax
from jax.experimental import pallas as pl
from jax.experimental.pallas import tpu as pltpu
from jax.experimental.pallas import tpu_sc as plsc

NC, NS = 2, 16
NW = NC * NS
NROW = 1024
DUMMY = 1008
D = 128
CH = 128
RPT = NROW // NS


def _make_sc_agg(E, filter_dst):
    EPW = E // NW
    NCH = (EPW + CH - 1) // CH
    KP = NCH * CH

    mesh = plsc.VectorSubcoreMesh(
        core_axis_name="c", subcore_axis_name="s",
        num_cores=NC, num_subcores=NS)
    out_type = (
        jax.ShapeDtypeStruct((NC, NROW, D), jnp.float32),
        jax.ShapeDtypeStruct((NC, NS, 8, 128), jnp.float32),
    )
    scratch = []
    if filter_dst:
        scratch += [pltpu.VMEM((EPW,), jnp.int32),
                    pltpu.VMEM((EPW,), jnp.int32)]
    scratch += [
        pltpu.VMEM((KP,), jnp.int32),
        pltpu.VMEM((KP,), jnp.int32),
        pltpu.VMEM((CH,), jnp.int32),
        pltpu.VMEM((CH,), jnp.int32),
        pltpu.VMEM((CH, D), jnp.float32),
        pltpu.VMEM((8, 128), jnp.float32),
        pltpu.VMEM_SHARED((NROW, D), jnp.float32),
        pltpu.SemaphoreType.DMA,
    ]

    def body(src_hbm, dst_hbm, table_hbm, sum_out, cnt_out, *sc):
        if filter_dst:
            src_loc, dst_loc = sc[0], sc[1]
            sc = sc[2:]
        src_keep, dst_keep, src_stage, dst_stage, rows, cnt_loc, acc, sem = sc
        cid = lax.axis_index("c")
        sid = lax.axis_index("s")
        wid = cid * NS + sid
        zf = jnp.zeros((16,), jnp.float32)
        zi = jnp.zeros((16,), jnp.int32)
        ones = jnp.ones((16,), jnp.float32)
        dums = jnp.full((16,), DUMMY, jnp.int32)
        iota = lax.iota(jnp.int32, 16)

        @pl.loop(0, RPT)
        def _zero(i):
            for j in range(D // 16):
                rows[i, pl.ds(j * 16, 16)] = zf

        for i in range(8):
            for j in range(8):
                cnt_loc[i, pl.ds(j * 16, 16)] = zf

        pltpu.sync_copy(rows.at[pl.ds(0, RPT)], acc.at[pl.ds(sid * RPT, RPT)])

        base = wid * EPW
        if filter_dst:
            pltpu.sync_copy(src_hbm.at[pl.ds(base, EPW)], src_loc)
            pltpu.sync_copy(dst_hbm.at[pl.ds(base, EPW)], dst_loc)
        else:
            pltpu.sync_copy(src_hbm.at[pl.ds(base, EPW)], src_keep.at[pl.ds(0, EPW)])
            pltpu.sync_copy(dst_hbm.at[pl.ds(base, EPW)], dst_keep.at[pl.ds(0, EPW)])

        plsc.subcore_barrier()

        if filter_dst:
            def cbody(i, ptr):
                s16 = src_loc[pl.ds(i * 16, 16)]
                d16 = dst_loc[pl.ds(i * 16, 16)]
                keep = d16 < 1000
                cs = plsc.cumsum(keep.astype(jnp.int32))
                pos = jnp.maximum(ptr + cs - 1, 0)
                plsc.store_scatter(src_keep, [pos], s16, mask=keep)
                plsc.store_scatter(dst_keep, [pos], d16, mask=keep)
                dsafe = jnp.where(keep, d16, DUMMY)
                plsc.addupdate_scatter(
                    cnt_loc, [dsafe >> 7, dsafe & 127], ones, mask=keep)
                return ptr + jnp.max(cs)

            nkeep = pl.loop(0, EPW // 16, init_carry=jnp.int32(0))(cbody)
        else:
            nkeep = EPW

        base0 = (nkeep // CH) * CH
        for j in range(CH // 16):
            idx = base0 + j * 16 + iota
            m = idx >= nkeep
            plsc.store_scatter(src_keep, [idx], zi, mask=m)
            plsc.store_scatter(dst_keep, [idx], dums, mask=m)

        if not filter_dst:
            @pl.loop(0, (EPW + 15) // 16)
            def _hist(i):
                d16 = dst_keep[pl.ds(i * 16, 16)]
                plsc.addupdate_scatter(cnt_loc, [d16 >> 7, d16 & 127], ones)

        nch = (nkeep + CH - 1) // CH

        @pl.loop(0, nch)
        def _chunk(g):
            for j in range(CH // 16):
                src_stage[pl.ds(j * 16, 16)] = src_keep[pl.ds(g * CH + j * 16, 16)]
                dst_stage[pl.ds(j * 16, 16)] = dst_keep[pl.ds(g * CH + j * 16, 16)]
            pltpu.async_copy(table_hbm.at[src_stage], rows, sem).wait()
            pltpu.sync_copy(rows, acc.at[dst_stage], add=True)

        plsc.subcore_barrier()
        pltpu.sync_copy(acc.at[pl.ds(sid * RPT, RPT)],
                        sum_out.at[cid, pl.ds(sid * RPT, RPT)])
        pltpu.sync_copy(cnt_loc, cnt_out.at[cid, sid])

    return pl.kernel(
        body, out_type=out_type, mesh=mesh, scratch_types=scratch,
        compiler_params=pltpu.CompilerParams(needs_layout_passes=False))


_E0, _E1 = 320000, 160000
_agg_l0 = _make_sc_agg(_E0, filter_dst=True)
_agg_l1 = _make_sc_agg(_E1, filter_dst=False)


def _dense0_body(sums, cnts, xd, wl, b, wr, out):
    s = sums[0] + sums[1]
    ccol = jnp.sum(cnts[...], axis=1, keepdims=True)
    agg = s * (1.0 / jnp.maximum(ccol, 1.0))
    h = jnp.dot(agg, wl[...], preferred_element_type=jnp.float32)
    h = h + b[...] + jnp.dot(xd[...], wr[...], preferred_element_type=jnp.float32)
    out[...] = jnp.maximum(h, 0.0)


def _dense1_body(sums, cnts, h0, wl, b, wr, wlin, blin, l0, l1, out):
    s = sums[0] + sums[1]
    ccol = jnp.sum(cnts[...], axis=1, keepdims=True)
    agg = s * (1.0 / jnp.maximum(ccol, 1.0))
    h = jnp.dot(agg, wl[...], preferred_element_type=jnp.float32)
    h = h + b[...] + jnp.dot(h0[...], wr[...], preferred_element_type=jnp.float32)
    h = jnp.maximum(h, 0.0)
    w2 = wlin[...]
    p0 = jnp.sum(h * w2[0:1, :], axis=1, keepdims=True)
    p1 = jnp.sum(h * w2[1:2, :], axis=1, keepdims=True)
    rowid = lax.broadcasted_iota(jnp.int32, (NROW, 512), 0)
    m0 = rowid == l0[...]
    m1 = rowid == l1[...]
    res = (jnp.sum(jnp.where(m0, p0, 0.0), axis=0)
           + jnp.sum(jnp.where(m1, p1, 0.0), axis=0))
    out[...] = res + blin[...][0]


def kernel(x, edge_index_0, edge_index_1, link, n_id,
           W0_l, b0_l, W0_r, W1_l, b1_l, W1_r, W_lin, b_lin):
    f32 = jnp.float32
    sums0, cnts0 = _agg_l0(edge_index_0[0], edge_index_0[1], x)
    h0 = pl.pallas_call(
        _dense0_body,
        out_shape=jax.ShapeDtypeStruct((NROW, D), f32),
    )(sums0, cnts0.reshape(NW, NROW).T, x[:NROW], W0_l,
      b0_l.reshape(1, D), W0_r)
    sums1, cnts1 = _agg_l1(edge_index_1[0], edge_index_1[1], h0)
    out = pl.pallas_call(
        _dense1_body,
        out_shape=jax.ShapeDtypeStruct((512,), f32),
    )(sums1, cnts1.reshape(NW, NROW).T, h0, W1_l,
      b1_l.reshape(1, D), W1_r, W_lin[:, 0].reshape(2, D),
      b_lin.reshape(1,), link[:, 0].reshape(1, 512), link[:, 1].reshape(1, 512))
    return out
